# baseline (speedup 1.0000x reference)
import functools

import jax
import jax.numpy as jnp
from jax import lax
from jax.experimental import pallas as pl
from jax.experimental.pallas import tpu as pltpu
from jax.experimental.pallas import tpu_sc as plsc


def _transpose_body(xt_ref, o_ref):
    x = xt_ref[...]
    d = x.shape[0]
    ii = lax.broadcasted_iota(jnp.int32, (d, d), 0)
    jj = lax.broadcasted_iota(jnp.int32, (d, d), 1)
    eye = (ii == jj).astype(jnp.float32)
    o_ref[...] = lax.dot_general(
        x, eye, (((0,), (0,)), ((), ())),
        preferred_element_type=jnp.float32,
    )


@functools.lru_cache(maxsize=None)
def _make_transpose(V, D):
    RBLK = 4096
    grid = (V + RBLK - 1) // RBLK
    return pl.pallas_call(
        _transpose_body,
        grid=(grid,),
        in_specs=[pl.BlockSpec((D, RBLK), lambda i: (0, i))],
        out_specs=pl.BlockSpec((RBLK, D), lambda i: (i, 0)),
        out_shape=jax.ShapeDtypeStruct((V, D), jnp.float32),
    )


@functools.lru_cache(maxsize=None)
def _make_gather(V, D, B, NC, NS):
    NW = NC * NS
    b_per_w = B // NW
    mesh = plsc.VectorSubcoreMesh(core_axis_name="c", subcore_axis_name="s")

    @functools.partial(
        pl.kernel,
        mesh=mesh,
        out_type=jax.ShapeDtypeStruct((B, D), jnp.float32),
        scratch_types=[
            pltpu.VMEM((b_per_w,), jnp.int32),
            pltpu.VMEM((b_per_w, D), jnp.float32),
            pltpu.SemaphoreType.DMA,
        ],
    )
    def gather(idx_hbm, table_hbm, out_hbm, idx_v, rows_v, sem):
        wid = lax.axis_index("s") * NC + lax.axis_index("c")
        base = wid * b_per_w
        table3 = table_hbm.reshape(V // 8, 8, D)
        pltpu.sync_copy(idx_hbm.at[wid], idx_v)

        def body(g, _):
            vec = idx_v[pl.ds(g * 16, 16)]
            for k in range(16):
                tid = vec[k]
                pltpu.async_copy(
                    table3.at[tid >> 3, tid & 7],
                    rows_v.at[g * 16 + k],
                    sem,
                )
            return 0

        lax.fori_loop(0, b_per_w // 16, body, 0)
        pltpu.make_async_copy(
            table_hbm.at[pl.ds(0, b_per_w)], rows_v, sem
        ).wait()
        pltpu.sync_copy(rows_v, out_hbm.at[pl.ds(base, b_per_w)])

    return gather


def _mlp_body(e_ref, w1_ref, b1_ref, w2_ref, b2_ref, o_ref):
    h = jnp.dot(e_ref[...], w1_ref[...], preferred_element_type=jnp.float32)
    h = jnp.maximum(h + b1_ref[...], 0.0)
    logit = jnp.sum(h * w2_ref[...], axis=1, keepdims=True) + b2_ref[...]
    o_ref[...] = 1.0 / (1.0 + jnp.exp(-logit))


@functools.lru_cache(maxsize=None)
def _make_mlp(B, H, F):
    BLK = 2048
    return pl.pallas_call(
        _mlp_body,
        grid=(B // BLK,),
        in_specs=[
            pl.BlockSpec((BLK, H), lambda i: (i, 0)),
            pl.BlockSpec((H, F), lambda i: (0, 0)),
            pl.BlockSpec((1, F), lambda i: (0, 0)),
            pl.BlockSpec((1, F), lambda i: (0, 0)),
            pl.BlockSpec((1, 1), lambda i: (0, 0)),
        ],
        out_specs=pl.BlockSpec((BLK, 1), lambda i: (i, 0)),
        out_shape=jax.ShapeDtypeStruct((B, 1), jnp.float32),
    )


def kernel(tool_token, table, W1, b1, W2, b2):
    B = tool_token.shape[0]
    V, D = table.shape
    H, F = W1.shape
    info = plsc.get_sparse_core_info()
    NC, NS = info.num_cores, info.num_subcores
    NW = NC * NS
    b_per_w = B // NW
    idx = tool_token.astype(jnp.int32).reshape(NW, b_per_w)
    table_rm = _make_transpose(V, D)(table.T)
    emb = _make_gather(V, D, B, NC, NS)(idx, table_rm)
    out = _make_mlp(B, H, F)(
        emb,
        W1,
        b1.reshape(1, F),
        W2.reshape(1, F),
        b2.reshape(1, 1),
    )
    return out


# speedup vs baseline: 1.2713x; 1.2713x over previous
import functools

import jax
import jax.numpy as jnp
from jax import lax
from jax.experimental import pallas as pl
from jax.experimental.pallas import tpu as pltpu
from jax.experimental.pallas import tpu_sc as plsc


def _transpose_body(xt_ref, o_ref):
    o_ref[...] = xt_ref[...].T


@functools.lru_cache(maxsize=None)
def _make_transpose(V, D):
    RBLK = 8192
    grid = (V + RBLK - 1) // RBLK
    return pl.pallas_call(
        _transpose_body,
        grid=(grid,),
        in_specs=[pl.BlockSpec((D, RBLK), lambda i: (0, i))],
        out_specs=pl.BlockSpec((RBLK, D), lambda i: (i, 0)),
        out_shape=jax.ShapeDtypeStruct((V, D), jnp.float32),
    )


@functools.lru_cache(maxsize=None)
def _make_gather(V, D, B, NC, NS):
    NW = NC * NS
    b_per_w = B // NW
    mesh = plsc.VectorSubcoreMesh(core_axis_name="c", subcore_axis_name="s")

    @functools.partial(
        pl.kernel,
        mesh=mesh,
        out_type=jax.ShapeDtypeStruct((B, D), jnp.float32),
        scratch_types=[
            pltpu.VMEM((b_per_w,), jnp.int32),
            pltpu.VMEM((b_per_w, D), jnp.float32),
            pltpu.SemaphoreType.DMA,
        ],
    )
    def gather(idx_hbm, table_hbm, out_hbm, idx_v, rows_v, sem):
        wid = lax.axis_index("s") * NC + lax.axis_index("c")
        base = wid * b_per_w
        table3 = table_hbm.reshape(V // 8, 8, D)
        pltpu.sync_copy(idx_hbm.at[wid], idx_v)

        def body(g, _):
            vec = idx_v[pl.ds(g * 16, 16)]
            for k in range(16):
                tid = vec[k]
                pltpu.async_copy(
                    table3.at[tid >> 3, tid & 7],
                    rows_v.at[g * 16 + k],
                    sem,
                )
            return 0

        lax.fori_loop(0, b_per_w // 16, body, 0)
        pltpu.make_async_copy(
            table_hbm.at[pl.ds(0, b_per_w)], rows_v, sem
        ).wait()
        pltpu.sync_copy(rows_v, out_hbm.at[pl.ds(base, b_per_w)])

    return gather


def _mlp_body(e_ref, w1_ref, b1_ref, w2_ref, b2_ref, o_ref):
    h = jnp.dot(e_ref[...], w1_ref[...], preferred_element_type=jnp.float32)
    h = jnp.maximum(h + b1_ref[...], 0.0)
    logit = jnp.sum(h * w2_ref[...], axis=1, keepdims=True) + b2_ref[...]
    o_ref[...] = 1.0 / (1.0 + jnp.exp(-logit))


@functools.lru_cache(maxsize=None)
def _make_mlp(B, H, F):
    BLK = 2048
    return pl.pallas_call(
        _mlp_body,
        grid=(B // BLK,),
        in_specs=[
            pl.BlockSpec((BLK, H), lambda i: (i, 0)),
            pl.BlockSpec((H, F), lambda i: (0, 0)),
            pl.BlockSpec((1, F), lambda i: (0, 0)),
            pl.BlockSpec((1, F), lambda i: (0, 0)),
            pl.BlockSpec((1, 1), lambda i: (0, 0)),
        ],
        out_specs=pl.BlockSpec((BLK, 1), lambda i: (i, 0)),
        out_shape=jax.ShapeDtypeStruct((B, 1), jnp.float32),
    )


def kernel(tool_token, table, W1, b1, W2, b2):
    B = tool_token.shape[0]
    V, D = table.shape
    H, F = W1.shape
    info = plsc.get_sparse_core_info()
    NC, NS = info.num_cores, info.num_subcores
    NW = NC * NS
    b_per_w = B // NW
    idx = tool_token.astype(jnp.int32).reshape(NW, b_per_w)
    table_rm = _make_transpose(V, D)(table.T)
    emb = _make_gather(V, D, B, NC, NS)(idx, table_rm)
    out = _make_mlp(B, H, F)(
        emb,
        W1,
        b1.reshape(1, F),
        W2.reshape(1, F),
        b2.reshape(1, 1),
    )
    return out


# speedup vs baseline: 1.3668x; 1.0752x over previous
import functools

import jax
import jax.numpy as jnp
from jax import lax
from jax.experimental import pallas as pl
from jax.experimental.pallas import tpu as pltpu
from jax.experimental.pallas import tpu_sc as plsc


def _transpose_body(xt_ref, o_ref):
    o_ref[...] = xt_ref[...].T


@functools.lru_cache(maxsize=None)
def _make_transpose(V, D):
    RBLK = 16384
    grid = (V + RBLK - 1) // RBLK
    return pl.pallas_call(
        _transpose_body,
        grid=(grid,),
        in_specs=[pl.BlockSpec((D, RBLK), lambda i: (0, i))],
        out_specs=pl.BlockSpec((RBLK, D), lambda i: (i, 0)),
        out_shape=jax.ShapeDtypeStruct((V, D), jnp.float32),
    )


@functools.lru_cache(maxsize=None)
def _make_gather(V, D, B, NC, NS):
    NW = NC * NS
    b_per_w = B // NW
    mesh = plsc.VectorSubcoreMesh(core_axis_name="c", subcore_axis_name="s")

    @functools.partial(
        pl.kernel,
        mesh=mesh,
        out_type=jax.ShapeDtypeStruct((B, D), jnp.float32),
        scratch_types=[
            pltpu.VMEM((b_per_w,), jnp.int32),
            pltpu.VMEM((b_per_w, D), jnp.float32),
            pltpu.SemaphoreType.DMA,
        ],
    )
    def gather(idx_hbm, table_hbm, out_hbm, idx_v, rows_v, sem):
        wid = lax.axis_index("s") * NC + lax.axis_index("c")
        base = wid * b_per_w
        table3 = table_hbm.reshape(V // 8, 8, D)
        pltpu.sync_copy(idx_hbm.at[wid], idx_v)

        def body(g, _):
            vec = idx_v[pl.ds(g * 16, 16)]
            for k in range(16):
                tid = vec[k]
                pltpu.async_copy(
                    table3.at[tid >> 3, tid & 7],
                    rows_v.at[g * 16 + k],
                    sem,
                )
            return 0

        lax.fori_loop(0, b_per_w // 16, body, 0)
        pltpu.make_async_copy(
            table_hbm.at[pl.ds(0, b_per_w)], rows_v, sem
        ).wait()
        pltpu.sync_copy(rows_v, out_hbm.at[pl.ds(base, b_per_w)])

    return gather


def _mlp_body(e_ref, w1_ref, b1_ref, w2_ref, b2_ref, o_ref):
    h = jnp.dot(e_ref[...], w1_ref[...], preferred_element_type=jnp.float32)
    h = jnp.maximum(h + b1_ref[...], 0.0)
    logit = jnp.sum(h * w2_ref[...], axis=1, keepdims=True) + b2_ref[...]
    o_ref[...] = 1.0 / (1.0 + jnp.exp(-logit))


@functools.lru_cache(maxsize=None)
def _make_mlp(B, H, F):
    BLK = 2048
    return pl.pallas_call(
        _mlp_body,
        grid=(B // BLK,),
        in_specs=[
            pl.BlockSpec((BLK, H), lambda i: (i, 0)),
            pl.BlockSpec((H, F), lambda i: (0, 0)),
            pl.BlockSpec((1, F), lambda i: (0, 0)),
            pl.BlockSpec((1, F), lambda i: (0, 0)),
            pl.BlockSpec((1, 1), lambda i: (0, 0)),
        ],
        out_specs=pl.BlockSpec((BLK, 1), lambda i: (i, 0)),
        out_shape=jax.ShapeDtypeStruct((B, 1), jnp.float32),
    )


def kernel(tool_token, table, W1, b1, W2, b2):
    B = tool_token.shape[0]
    V, D = table.shape
    H, F = W1.shape
    info = plsc.get_sparse_core_info()
    NC, NS = info.num_cores, info.num_subcores
    NW = NC * NS
    b_per_w = B // NW
    idx = tool_token.astype(jnp.int32).reshape(NW, b_per_w)
    table_rm = _make_transpose(V, D)(table.T)
    emb = _make_gather(V, D, B, NC, NS)(idx, table_rm)
    out = _make_mlp(B, H, F)(
        emb,
        W1,
        b1.reshape(1, F),
        W2.reshape(1, F),
        b2.reshape(1, 1),
    )
    return out


# speedup vs baseline: 1.3846x; 1.0130x over previous
import functools

import jax
import jax.numpy as jnp
from jax import lax
from jax.experimental import pallas as pl
from jax.experimental.pallas import tpu as pltpu
from jax.experimental.pallas import tpu_sc as plsc


def _transpose_body(xt_ref, o_ref):
    o_ref[...] = xt_ref[...].T


@functools.lru_cache(maxsize=None)
def _make_transpose(V, D):
    RBLK = 32768
    grid = (V + RBLK - 1) // RBLK
    return pl.pallas_call(
        _transpose_body,
        grid=(grid,),
        in_specs=[pl.BlockSpec((D, RBLK), lambda i: (0, i))],
        out_specs=pl.BlockSpec((RBLK, D), lambda i: (i, 0)),
        out_shape=jax.ShapeDtypeStruct((V, D), jnp.float32),
    )


@functools.lru_cache(maxsize=None)
def _make_gather(V, D, B, NC, NS):
    NW = NC * NS
    b_per_w = B // NW
    mesh = plsc.VectorSubcoreMesh(core_axis_name="c", subcore_axis_name="s")

    @functools.partial(
        pl.kernel,
        mesh=mesh,
        out_type=jax.ShapeDtypeStruct((B, D), jnp.float32),
        scratch_types=[
            pltpu.VMEM((b_per_w,), jnp.int32),
            pltpu.VMEM((b_per_w, D), jnp.float32),
            pltpu.SemaphoreType.DMA,
        ],
    )
    def gather(idx_hbm, table_hbm, out_hbm, idx_v, rows_v, sem):
        wid = lax.axis_index("s") * NC + lax.axis_index("c")
        base = wid * b_per_w
        table3 = table_hbm.reshape(V // 8, 8, D)
        pltpu.sync_copy(idx_hbm.at[wid], idx_v)

        def body(g, _):
            vec = idx_v[pl.ds(g * 16, 16)]
            for k in range(16):
                tid = vec[k]
                pltpu.async_copy(
                    table3.at[tid >> 3, tid & 7],
                    rows_v.at[g * 16 + k],
                    sem,
                )
            return 0

        lax.fori_loop(0, b_per_w // 16, body, 0)
        pltpu.make_async_copy(
            table_hbm.at[pl.ds(0, b_per_w)], rows_v, sem
        ).wait()
        pltpu.sync_copy(rows_v, out_hbm.at[pl.ds(base, b_per_w)])

    return gather


def _mlp_body(e_ref, w1_ref, b1_ref, w2_ref, b2_ref, o_ref):
    h = jnp.dot(e_ref[...], w1_ref[...], preferred_element_type=jnp.float32)
    h = jnp.maximum(h + b1_ref[...], 0.0)
    logit = jnp.sum(h * w2_ref[...], axis=1, keepdims=True) + b2_ref[...]
    o_ref[...] = 1.0 / (1.0 + jnp.exp(-logit))


@functools.lru_cache(maxsize=None)
def _make_mlp(B, H, F):
    BLK = 2048
    return pl.pallas_call(
        _mlp_body,
        grid=(B // BLK,),
        in_specs=[
            pl.BlockSpec((BLK, H), lambda i: (i, 0)),
            pl.BlockSpec((H, F), lambda i: (0, 0)),
            pl.BlockSpec((1, F), lambda i: (0, 0)),
            pl.BlockSpec((1, F), lambda i: (0, 0)),
            pl.BlockSpec((1, 1), lambda i: (0, 0)),
        ],
        out_specs=pl.BlockSpec((BLK, 1), lambda i: (i, 0)),
        out_shape=jax.ShapeDtypeStruct((B, 1), jnp.float32),
    )


def kernel(tool_token, table, W1, b1, W2, b2):
    B = tool_token.shape[0]
    V, D = table.shape
    H, F = W1.shape
    info = plsc.get_sparse_core_info()
    NC, NS = info.num_cores, info.num_subcores
    NW = NC * NS
    b_per_w = B // NW
    idx = tool_token.astype(jnp.int32).reshape(NW, b_per_w)
    table_rm = _make_transpose(V, D)(table.T)
    emb = _make_gather(V, D, B, NC, NS)(idx, table_rm)
    out = _make_mlp(B, H, F)(
        emb,
        W1,
        b1.reshape(1, F),
        W2.reshape(1, F),
        b2.reshape(1, 1),
    )
    return out
